# baseline (speedup 1.0000x reference)
import functools

import jax
import jax.numpy as jnp
from jax import lax
from jax.experimental import pallas as pl
from jax.experimental.pallas import tpu as pltpu
from jax.experimental.pallas import tpu_sc as plsc

_NC = 2
_NS = 16
_NW = _NC * _NS

_EB = 3200


def _store_slots(ref, val, h):
    grp = val.shape[0] // 8
    for g in range(8):
        ref[:, g * h:(g + 1) * h] = val[g * grp:(g + 1) * grp, :]


def _load_slots(val_w, h):
    return jnp.concatenate(
        [val_w[:, g * h:(g + 1) * h] for g in range(8)], axis=0)


def _in_proj_body(x_ref, w_ref, b_ref, r_ref, bc_ref, wm_ref, bm_ref,
                  o_ref, hr_ref, hm_ref):
    h = w_ref.shape[1]
    hv = (
        jnp.dot(x_ref[...], w_ref[...], preferred_element_type=jnp.float32)
        + b_ref[...]
    )
    _store_slots(o_ref, hv, h)
    hr = (
        jnp.dot(hv, r_ref[...], preferred_element_type=jnp.float32)
        + bc_ref[...]
    )
    _store_slots(hr_ref, hr, h)
    hm = jnp.maximum(
        jnp.dot(hv, wm_ref[...], preferred_element_type=jnp.float32)
        + bm_ref[...], 0.0)
    _store_slots(hm_ref, hm, h)


def _edge_msg_body(et_ref, hs_ref, wf_ref, bf_ref, o_ref):
    h = et_ref.shape[0]
    hh = h * h
    ev = et_ref[...].T
    hs = _load_slots(hs_ref[...], h)
    kern = (
        jnp.dot(ev, wf_ref[...], preferred_element_type=jnp.float32)
        + bf_ref[...]
    )
    rf = lax.broadcasted_iota(jnp.int32, (h, hh), 0)
    rj = lax.broadcasted_iota(jnp.int32, (h, hh), 1) // h
    rmat = jnp.where(rf == rj, 1.0, 0.0).astype(jnp.float32)
    hrep = jnp.dot(hs, rmat, preferred_element_type=jnp.float32)
    sj = lax.broadcasted_iota(jnp.int32, (hh, h), 0) % h
    so = lax.broadcasted_iota(jnp.int32, (hh, h), 1)
    smat = jnp.where(sj == so, 1.0, 0.0).astype(jnp.float32)
    msg = jnp.dot(kern * hrep, smat, preferred_element_type=jnp.float32)
    _store_slots(o_ref, msg, h)


def _update_out_body(h_ref, p_ref, r_ref, bc_ref, wm_ref, bm_ref,
                     wo_ref, bo_ref, o_ref):
    h = r_ref.shape[0]
    half = p_ref.shape[0] // 2
    hv = _load_slots(h_ref[...], h)
    pv = p_ref[...]
    agg = _load_slots(pv[:half] + pv[half:], h)
    hg = jnp.maximum(
        agg + jnp.dot(hv, r_ref[...], preferred_element_type=jnp.float32)
        + bc_ref[...], 0.0)
    hm = jnp.maximum(
        jnp.dot(hv, wm_ref[...], preferred_element_type=jnp.float32)
        + bm_ref[...], 0.0)
    hn = hg + hm + hv
    o_ref[...] = (
        jnp.dot(hn, wo_ref[...], preferred_element_type=jnp.float32)
        + bo_ref[...]
    )


def _full(shape):
    return pl.BlockSpec(shape, lambda i: tuple(0 for _ in shape))


def _in_proj(x, w, b, r, bc, wm, bm):
    n, d = x.shape
    h = w.shape[1]
    wn = n * h // 128
    wspec = _full((wn, 128))
    return pl.pallas_call(
        _in_proj_body,
        grid=(1,),
        in_specs=[
            _full((n, d)),
            _full((d, h)),
            _full((1, h)),
            _full((h, h)),
            _full((1, h)),
            _full((h, h)),
            _full((1, h)),
        ],
        out_specs=[wspec, wspec, wspec],
        out_shape=[jax.ShapeDtypeStruct((wn, 128), jnp.float32)] * 3,
    )(x, w, b.reshape(1, h), r, bc.reshape(1, h), wm, bm.reshape(1, h))


def _edge_msg(et, h_src_w, wf, bf):
    d_edge, ecount = et.shape
    h = d_edge
    hh = h * h
    wb = _EB * h // 128
    return pl.pallas_call(
        _edge_msg_body,
        grid=(ecount // _EB,),
        in_specs=[
            pl.BlockSpec((d_edge, _EB), lambda i: (0, i)),
            pl.BlockSpec((wb, 128), lambda i: (i, 0)),
            _full((d_edge, hh)),
            _full((1, hh)),
        ],
        out_specs=pl.BlockSpec((wb, 128), lambda i: (i, 0)),
        out_shape=jax.ShapeDtypeStruct((ecount * h // 128, 128), jnp.float32),
    )(et, h_src_w, wf, bf.reshape(1, hh))


def _update_out(h_w, parts_w, r, bc, wm, bm, wo, bo):
    h = r.shape[0]
    n = h_w.shape[0] * 128 // h
    o = wo.shape[1]
    return pl.pallas_call(
        _update_out_body,
        grid=(1,),
        in_specs=[
            _full(h_w.shape),
            _full(parts_w.shape),
            _full((h, h)),
            _full((1, h)),
            _full((h, h)),
            _full((1, h)),
            _full((h, o)),
            _full((1, o)),
        ],
        out_specs=_full((n, o)),
        out_shape=jax.ShapeDtypeStruct((n, o), jnp.float32),
    )(h_w, parts_w, r, bc.reshape(1, h), wm, bm.reshape(1, h),
      wo, bo.reshape(1, o))


@functools.cache
def _make_gather(n, ecount, h):
    epw = ecount // _NW
    mesh = plsc.VectorSubcoreMesh(core_axis_name="c", subcore_axis_name="s")

    @functools.partial(
        pl.kernel,
        out_type=jax.ShapeDtypeStruct((ecount, h), jnp.float32),
        mesh=mesh,
        compiler_params=pltpu.CompilerParams(use_tc_tiling_on_sc=False),
        scratch_types=[
            pltpu.VMEM((epw,), jnp.int32),
            pltpu.VMEM((epw, h), jnp.float32),
            pltpu.SemaphoreType.DMA,
        ],
    )
    def gather(h_hbm, idx_hbm, out_hbm, idx_v, rows_v, sem):
        wid = lax.axis_index("s") * _NC + lax.axis_index("c")
        base = wid * epw
        pltpu.sync_copy(idx_hbm.at[pl.ds(base, epw)], idx_v)
        pltpu.async_copy(h_hbm.at[idx_v], rows_v, sem).wait()
        pltpu.sync_copy(rows_v, out_hbm.at[pl.ds(base, epw)])

    return gather


@functools.cache
def _make_scatter(n, ecount, h):
    epw = ecount // _NW
    npt = n // _NS
    mesh = plsc.VectorSubcoreMesh(core_axis_name="c", subcore_axis_name="s")

    @functools.partial(
        pl.kernel,
        out_type=jax.ShapeDtypeStruct((_NC * n, h), jnp.float32),
        mesh=mesh,
        compiler_params=pltpu.CompilerParams(use_tc_tiling_on_sc=False),
        scratch_types=[
            pltpu.VMEM((epw,), jnp.int32),
            pltpu.VMEM((epw, h), jnp.float32),
            pltpu.VMEM((npt, h), jnp.float32),
            pltpu.VMEM_SHARED((n, h), jnp.float32),
            pltpu.SemaphoreType.DMA,
            pltpu.SemaphoreType.DMA,
        ],
    )
    def scatter(msg_hbm, dst_hbm, out_hbm, idx_v, rows_v, obuf, agg_sh,
                sem_i, sem_r):
        cid = lax.axis_index("c")
        sid = lax.axis_index("s")
        wid = cid * _NS + sid
        base = wid * epw

        cp_i = pltpu.async_copy(dst_hbm.at[pl.ds(base, epw)], idx_v, sem_i)
        cp_r = pltpu.async_copy(msg_hbm.at[pl.ds(base, epw)], rows_v, sem_r)

        def zero_row(i, _):
            obuf[i, :] = jnp.zeros((h,), jnp.float32)
            return 0
        lax.fori_loop(0, npt, zero_row, 0)
        pltpu.sync_copy(obuf, agg_sh.at[pl.ds(sid * npt, npt)])
        cp_i.wait()
        cp_r.wait()
        plsc.subcore_barrier()

        pltpu.sync_copy(rows_v, agg_sh.at[idx_v], add=True)
        plsc.subcore_barrier()

        pltpu.sync_copy(agg_sh.at[pl.ds(sid * npt, npt)], obuf)
        pltpu.sync_copy(obuf, out_hbm.at[pl.ds(cid * n + sid * npt, npt)])

    return scatter


@functools.cache
def _make_scatter_update_gather(n, ecount, h):
    ept = ecount // _NS
    q = ept // 5
    epw = ecount // _NW
    ga = q
    gb = epw - q
    npt = n // _NS
    mesh = plsc.VectorSubcoreMesh(core_axis_name="c", subcore_axis_name="s")

    @functools.partial(
        pl.kernel,
        out_type=(
            jax.ShapeDtypeStruct((n, h), jnp.float32),
            jax.ShapeDtypeStruct((ecount, h), jnp.float32),
        ),
        mesh=mesh,
        compiler_params=pltpu.CompilerParams(use_tc_tiling_on_sc=False),
        scratch_types=[
            pltpu.VMEM((5, q), jnp.int32),
            pltpu.VMEM((ga,), jnp.int32),
            pltpu.VMEM((gb,), jnp.int32),
            pltpu.VMEM((q, h), jnp.float32),
            pltpu.VMEM((gb, h), jnp.float32),
            pltpu.VMEM((npt, h), jnp.float32),
            pltpu.VMEM_SHARED((n, h), jnp.float32),
            pltpu.VMEM_SHARED((n, h), jnp.float32),
            pltpu.SemaphoreType.DMA,
            pltpu.SemaphoreType.DMA,
            pltpu.SemaphoreType.DMA,
            pltpu.SemaphoreType.DMA,
        ],
    )
    def fused(msg_hbm, dst_hbm, src_hbm, h_hbm, hr_hbm, hm_hbm,
              hnew_hbm, hsrc_hbm,
              idx_v, sidx_a, sidx_b, rows_a, rows_b, zbuf,
              agg_sh, h2_sh, sem_i, sem_a, sem_b, sem_g):
        cid = lax.axis_index("c")
        sid = lax.axis_index("s")
        ebase = sid * ept
        nbase = sid * npt
        gbase = (sid * _NC + cid) * epw

        cpi = [pltpu.async_copy(dst_hbm.at[pl.ds(ebase + k * q, q)],
                                idx_v.at[k], sem_i)
               for k in range(5)]
        cps_a = pltpu.async_copy(src_hbm.at[pl.ds(gbase, ga)], sidx_a, sem_g)
        cps_b = pltpu.async_copy(src_hbm.at[pl.ds(gbase + ga, gb)], sidx_b,
                                 sem_g)
        bufs = [rows_a, rows_b.at[pl.ds(0, q)]]
        sems = [sem_a, sem_b]
        cr = [pltpu.async_copy(msg_hbm.at[pl.ds(ebase, q)], bufs[0], sems[0]),
              pltpu.async_copy(msg_hbm.at[pl.ds(ebase + q, q)], bufs[1],
                               sems[1])]

        def zero_row(i, _):
            zbuf[i, :] = jnp.zeros((h,), jnp.float32)
            return 0
        lax.fori_loop(0, npt, zero_row, 0)
        pltpu.sync_copy(zbuf, agg_sh.at[pl.ds(nbase, npt)])
        for c in cpi:
            c.wait()
        plsc.subcore_barrier()

        for k in range(5):
            cr[k].wait()
            pltpu.sync_copy(bufs[k % 2], agg_sh.at[idx_v.at[k]], add=True)
            if k + 2 < 5:
                cr.append(pltpu.async_copy(
                    msg_hbm.at[pl.ds(ebase + (k + 2) * q, q)],
                    bufs[k % 2], sems[k % 2]))
        plsc.subcore_barrier()

        cu_h = pltpu.async_copy(h_hbm.at[pl.ds(nbase, npt)],
                                rows_a.at[pl.ds(0, npt)], sem_a)
        cu_r = pltpu.async_copy(hr_hbm.at[pl.ds(nbase, npt)],
                                rows_a.at[pl.ds(npt, npt)], sem_b)
        cu_m = pltpu.async_copy(hm_hbm.at[pl.ds(nbase, npt)],
                                rows_a.at[pl.ds(2 * npt, npt)], sem_i)
        pltpu.sync_copy(agg_sh.at[pl.ds(nbase, npt)],
                        rows_b.at[pl.ds(0, npt)])
        cu_h.wait()
        cu_r.wait()
        cu_m.wait()

        def upd_row(i, _):
            hg = jnp.maximum(rows_b[i, :] + rows_a[npt + i, :], 0.0)
            rows_b[npt + i, :] = hg + rows_a[2 * npt + i, :] + rows_a[i, :]
            return 0
        lax.fori_loop(0, npt, upd_row, 0)

        pltpu.sync_copy(rows_b.at[pl.ds(npt, npt)],
                        h2_sh.at[pl.ds(nbase, npt)])

        @pl.when(cid == 0)
        def _():
            pltpu.sync_copy(rows_b.at[pl.ds(npt, npt)],
                            hnew_hbm.at[pl.ds(nbase, npt)])

        cps_a.wait()
        cps_b.wait()
        plsc.subcore_barrier()

        g_a = pltpu.async_copy(h2_sh.at[sidx_a], rows_a, sem_a)
        g_b = pltpu.async_copy(h2_sh.at[sidx_b], rows_b, sem_b)
        g_a.wait()
        pltpu.sync_copy(rows_a, hsrc_hbm.at[pl.ds(gbase, ga)])
        g_b.wait()
        pltpu.sync_copy(rows_b, hsrc_hbm.at[pl.ds(gbase + ga, gb)])

    return fused


def kernel(x, edge_index, e, W_in, b_in,
           W_fgn_0, b_fgn_0, root_0, b_ecc_0, W_mlp_0, b_mlp_0,
           W_fgn_1, b_fgn_1, root_1, b_ecc_1, W_mlp_1, b_mlp_1,
           W_out, b_out):
    n = x.shape[0]
    ecount = edge_index.shape[1]
    h = W_in.shape[1]

    p = jnp.arange(ecount, dtype=jnp.int32)
    perm_e = (p // _EB) * _EB + (_EB // 8) * (p % 8) + (p % _EB) // 8
    idx_s = jnp.take(edge_index, perm_e, axis=1)
    idx_s = 8 * (idx_s % (n // 8)) + idx_s // (n // 8)
    src_s = idx_s[0]
    dst_s = idx_s[1]
    et = e.T

    gather = _make_gather(n, ecount, h)
    scatter = _make_scatter(n, ecount, h)
    fused = _make_scatter_update_gather(n, ecount, h)

    h1w, hr1w, hm1w = _in_proj(x, W_in, b_in,
                               root_0, b_ecc_0, W_mlp_0, b_mlp_0)
    h_src1 = gather(h1w.reshape(n, h), src_s)
    msg1w = _edge_msg(et, h_src1.reshape(ecount * h // 128, 128),
                      W_fgn_0, b_fgn_0)
    h2, h_src2 = fused(msg1w.reshape(ecount, h), dst_s, src_s,
                       h1w.reshape(n, h), hr1w.reshape(n, h),
                       hm1w.reshape(n, h))
    msg2w = _edge_msg(et, h_src2.reshape(ecount * h // 128, 128),
                      W_fgn_1, b_fgn_1)
    parts = scatter(msg2w.reshape(ecount, h), dst_s)
    return _update_out(h2.reshape(n * h // 128, 128),
                       parts.reshape(_NC * n * h // 128, 128),
                       root_1, b_ecc_1, W_mlp_1, b_mlp_1, W_out, b_out)


# speedup vs baseline: 2.0128x; 2.0128x over previous
import functools

import jax
import jax.numpy as jnp
from jax import lax
from jax.experimental import pallas as pl
from jax.experimental.pallas import tpu as pltpu
from jax.experimental.pallas import tpu_sc as plsc

_NC = 2
_NS = 16
_NW = _NC * _NS

_EB = 6400


def _store_slots(ref, val, h):
    grp = val.shape[0] // 8
    for g in range(8):
        ref[:, g * h:(g + 1) * h] = val[g * grp:(g + 1) * grp, :]


def _load_slots(val_w, h):
    return jnp.concatenate(
        [val_w[:, g * h:(g + 1) * h] for g in range(8)], axis=0)


def _in_proj_body(x_ref, w_ref, b_ref, r_ref, bc_ref, wm_ref, bm_ref,
                  o_ref, hr_ref, hm_ref):
    h = w_ref.shape[1]
    hv = (
        jnp.dot(x_ref[...], w_ref[...], preferred_element_type=jnp.float32)
        + b_ref[...]
    )
    _store_slots(o_ref, hv, h)
    hr = (
        jnp.dot(hv, r_ref[...], preferred_element_type=jnp.float32)
        + bc_ref[...]
    )
    _store_slots(hr_ref, hr, h)
    hm = jnp.maximum(
        jnp.dot(hv, wm_ref[...], preferred_element_type=jnp.float32)
        + bm_ref[...], 0.0)
    _store_slots(hm_ref, hm, h)


def _edge_msg_body(et_ref, hs_ref, wf_ref, bf_ref, o_ref):
    h = et_ref.shape[0]
    hh = h * h
    ev = et_ref[...].T
    hs = _load_slots(hs_ref[...], h)
    kern = (
        jnp.dot(ev, wf_ref[...], preferred_element_type=jnp.float32)
        + bf_ref[...]
    )
    rf = lax.broadcasted_iota(jnp.int32, (h, hh), 0)
    rj = lax.broadcasted_iota(jnp.int32, (h, hh), 1) // h
    rmat = jnp.where(rf == rj, 1.0, 0.0).astype(jnp.float32)
    hrep = jnp.dot(hs, rmat, preferred_element_type=jnp.float32)
    sj = lax.broadcasted_iota(jnp.int32, (hh, h), 0) % h
    so = lax.broadcasted_iota(jnp.int32, (hh, h), 1)
    smat = jnp.where(sj == so, 1.0, 0.0).astype(jnp.float32)
    msg = jnp.dot(kern * hrep, smat, preferred_element_type=jnp.float32)
    _store_slots(o_ref, msg, h)


def _update_out_body(h_ref, p_ref, r_ref, bc_ref, wm_ref, bm_ref,
                     wo_ref, bo_ref, o_ref):
    h = r_ref.shape[0]
    half = p_ref.shape[0] // 2
    hv = _load_slots(h_ref[...], h)
    pv = p_ref[...]
    agg = _load_slots(pv[:half] + pv[half:], h)
    hg = jnp.maximum(
        agg + jnp.dot(hv, r_ref[...], preferred_element_type=jnp.float32)
        + bc_ref[...], 0.0)
    hm = jnp.maximum(
        jnp.dot(hv, wm_ref[...], preferred_element_type=jnp.float32)
        + bm_ref[...], 0.0)
    hn = hg + hm + hv
    o_ref[...] = (
        jnp.dot(hn, wo_ref[...], preferred_element_type=jnp.float32)
        + bo_ref[...]
    )


def _full(shape):
    return pl.BlockSpec(shape, lambda i: tuple(0 for _ in shape))


def _in_proj(x, w, b, r, bc, wm, bm):
    n, d = x.shape
    h = w.shape[1]
    wn = n * h // 128
    wspec = _full((wn, 128))
    return pl.pallas_call(
        _in_proj_body,
        grid=(1,),
        in_specs=[
            _full((n, d)),
            _full((d, h)),
            _full((1, h)),
            _full((h, h)),
            _full((1, h)),
            _full((h, h)),
            _full((1, h)),
        ],
        out_specs=[wspec, wspec, wspec],
        out_shape=[jax.ShapeDtypeStruct((wn, 128), jnp.float32)] * 3,
    )(x, w, b.reshape(1, h), r, bc.reshape(1, h), wm, bm.reshape(1, h))


def _edge_msg(et, h_src_w, wf, bf):
    d_edge, ecount = et.shape
    h = d_edge
    hh = h * h
    wb = _EB * h // 128
    return pl.pallas_call(
        _edge_msg_body,
        grid=(ecount // _EB,),
        in_specs=[
            pl.BlockSpec((d_edge, _EB), lambda i: (0, i)),
            pl.BlockSpec((wb, 128), lambda i: (i, 0)),
            _full((d_edge, hh)),
            _full((1, hh)),
        ],
        out_specs=pl.BlockSpec((wb, 128), lambda i: (i, 0)),
        out_shape=jax.ShapeDtypeStruct((ecount * h // 128, 128), jnp.float32),
    )(et, h_src_w, wf, bf.reshape(1, hh))


def _update_out(h_w, parts_w, r, bc, wm, bm, wo, bo):
    h = r.shape[0]
    n = h_w.shape[0] * 128 // h
    o = wo.shape[1]
    return pl.pallas_call(
        _update_out_body,
        grid=(1,),
        in_specs=[
            _full(h_w.shape),
            _full(parts_w.shape),
            _full((h, h)),
            _full((1, h)),
            _full((h, h)),
            _full((1, h)),
            _full((h, o)),
            _full((1, o)),
        ],
        out_specs=_full((n, o)),
        out_shape=jax.ShapeDtypeStruct((n, o), jnp.float32),
    )(h_w, parts_w, r, bc.reshape(1, h), wm, bm.reshape(1, h),
      wo, bo.reshape(1, o))


@functools.cache
def _make_gather(n, ecount, h):
    epw = ecount // _NW
    mesh = plsc.VectorSubcoreMesh(core_axis_name="c", subcore_axis_name="s")

    @functools.partial(
        pl.kernel,
        out_type=jax.ShapeDtypeStruct((ecount, h), jnp.float32),
        mesh=mesh,
        compiler_params=pltpu.CompilerParams(use_tc_tiling_on_sc=False),
        scratch_types=[
            pltpu.VMEM((epw,), jnp.int32),
            pltpu.VMEM((epw, h), jnp.float32),
            pltpu.SemaphoreType.DMA,
        ],
    )
    def gather(h_hbm, idx_hbm, out_hbm, idx_v, rows_v, sem):
        wid = lax.axis_index("s") * _NC + lax.axis_index("c")
        base = wid * epw
        pltpu.sync_copy(idx_hbm.at[pl.ds(base, epw)], idx_v)
        pltpu.async_copy(h_hbm.at[idx_v], rows_v, sem).wait()
        pltpu.sync_copy(rows_v, out_hbm.at[pl.ds(base, epw)])

    return gather


@functools.cache
def _make_scatter(n, ecount, h):
    epw = ecount // _NW
    npt = n // _NS
    mesh = plsc.VectorSubcoreMesh(core_axis_name="c", subcore_axis_name="s")

    @functools.partial(
        pl.kernel,
        out_type=jax.ShapeDtypeStruct((_NC * n, h), jnp.float32),
        mesh=mesh,
        compiler_params=pltpu.CompilerParams(use_tc_tiling_on_sc=False),
        scratch_types=[
            pltpu.VMEM((epw,), jnp.int32),
            pltpu.VMEM((epw, h), jnp.float32),
            pltpu.VMEM((npt, h), jnp.float32),
            pltpu.VMEM_SHARED((n, h), jnp.float32),
            pltpu.SemaphoreType.DMA,
            pltpu.SemaphoreType.DMA,
        ],
    )
    def scatter(msg_hbm, dst_hbm, out_hbm, idx_v, rows_v, obuf, agg_sh,
                sem_i, sem_r):
        cid = lax.axis_index("c")
        sid = lax.axis_index("s")
        wid = cid * _NS + sid
        base = wid * epw

        cp_i = pltpu.async_copy(dst_hbm.at[pl.ds(base, epw)], idx_v, sem_i)
        cp_r = pltpu.async_copy(msg_hbm.at[pl.ds(base, epw)], rows_v, sem_r)

        def zero_row(i, _):
            obuf[i, :] = jnp.zeros((h,), jnp.float32)
            return 0
        lax.fori_loop(0, npt, zero_row, 0)
        pltpu.sync_copy(obuf, agg_sh.at[pl.ds(sid * npt, npt)])
        cp_i.wait()
        cp_r.wait()
        plsc.subcore_barrier()

        pltpu.sync_copy(rows_v, agg_sh.at[idx_v], add=True)
        plsc.subcore_barrier()

        pltpu.sync_copy(agg_sh.at[pl.ds(sid * npt, npt)], obuf)
        pltpu.sync_copy(obuf, out_hbm.at[pl.ds(cid * n + sid * npt, npt)])

    return scatter


@functools.cache
def _make_scatter_update_gather(n, ecount, h):
    ept = ecount // _NS
    q = ept // 5
    epw = ecount // _NW
    ga = q
    gb = epw - q
    npt = n // _NS
    mesh = plsc.VectorSubcoreMesh(core_axis_name="c", subcore_axis_name="s")

    @functools.partial(
        pl.kernel,
        out_type=(
            jax.ShapeDtypeStruct((n, h), jnp.float32),
            jax.ShapeDtypeStruct((ecount, h), jnp.float32),
        ),
        mesh=mesh,
        compiler_params=pltpu.CompilerParams(use_tc_tiling_on_sc=False),
        scratch_types=[
            pltpu.VMEM((5, q), jnp.int32),
            pltpu.VMEM((ga,), jnp.int32),
            pltpu.VMEM((gb,), jnp.int32),
            pltpu.VMEM((q, h), jnp.float32),
            pltpu.VMEM((gb, h), jnp.float32),
            pltpu.VMEM((npt, h), jnp.float32),
            pltpu.VMEM_SHARED((n, h), jnp.float32),
            pltpu.VMEM_SHARED((n, h), jnp.float32),
            pltpu.SemaphoreType.DMA,
            pltpu.SemaphoreType.DMA,
            pltpu.SemaphoreType.DMA,
            pltpu.SemaphoreType.DMA,
        ],
    )
    def fused(msg_hbm, dst_hbm, src_hbm, h_hbm, hr_hbm, hm_hbm,
              hnew_hbm, hsrc_hbm,
              idx_v, sidx_a, sidx_b, rows_a, rows_b, zbuf,
              agg_sh, h2_sh, sem_i, sem_a, sem_b, sem_g):
        cid = lax.axis_index("c")
        sid = lax.axis_index("s")
        ebase = sid * ept
        nbase = sid * npt
        gbase = (sid * _NC + cid) * epw

        cpi = [pltpu.async_copy(dst_hbm.at[pl.ds(ebase + k * q, q)],
                                idx_v.at[k], sem_i)
               for k in range(5)]
        cps_a = pltpu.async_copy(src_hbm.at[pl.ds(gbase, ga)], sidx_a, sem_g)
        cps_b = pltpu.async_copy(src_hbm.at[pl.ds(gbase + ga, gb)], sidx_b,
                                 sem_g)
        bufs = [rows_a, rows_b.at[pl.ds(0, q)]]
        sems = [sem_a, sem_b]
        cr = [pltpu.async_copy(msg_hbm.at[pl.ds(ebase, q)], bufs[0], sems[0]),
              pltpu.async_copy(msg_hbm.at[pl.ds(ebase + q, q)], bufs[1],
                               sems[1])]

        def zero_row(i, _):
            zbuf[i, :] = jnp.zeros((h,), jnp.float32)
            return 0
        lax.fori_loop(0, npt, zero_row, 0)
        pltpu.sync_copy(zbuf, agg_sh.at[pl.ds(nbase, npt)])
        for c in cpi:
            c.wait()
        plsc.subcore_barrier()

        for k in range(5):
            cr[k].wait()
            pltpu.sync_copy(bufs[k % 2], agg_sh.at[idx_v.at[k]], add=True)
            if k + 2 < 5:
                cr.append(pltpu.async_copy(
                    msg_hbm.at[pl.ds(ebase + (k + 2) * q, q)],
                    bufs[k % 2], sems[k % 2]))
        plsc.subcore_barrier()

        cu_h = pltpu.async_copy(h_hbm.at[pl.ds(nbase, npt)],
                                rows_a.at[pl.ds(0, npt)], sem_a)
        cu_r = pltpu.async_copy(hr_hbm.at[pl.ds(nbase, npt)],
                                rows_a.at[pl.ds(npt, npt)], sem_b)
        cu_m = pltpu.async_copy(hm_hbm.at[pl.ds(nbase, npt)],
                                rows_a.at[pl.ds(2 * npt, npt)], sem_i)
        pltpu.sync_copy(agg_sh.at[pl.ds(nbase, npt)],
                        rows_b.at[pl.ds(0, npt)])
        cu_h.wait()
        cu_r.wait()
        cu_m.wait()

        def upd_row(i, _):
            hg = jnp.maximum(rows_b[i, :] + rows_a[npt + i, :], 0.0)
            rows_b[npt + i, :] = hg + rows_a[2 * npt + i, :] + rows_a[i, :]
            return 0
        lax.fori_loop(0, npt, upd_row, 0)

        pltpu.sync_copy(rows_b.at[pl.ds(npt, npt)],
                        h2_sh.at[pl.ds(nbase, npt)])

        @pl.when(cid == 0)
        def _():
            pltpu.sync_copy(rows_b.at[pl.ds(npt, npt)],
                            hnew_hbm.at[pl.ds(nbase, npt)])

        cps_a.wait()
        cps_b.wait()
        plsc.subcore_barrier()

        g_a = pltpu.async_copy(h2_sh.at[sidx_a], rows_a, sem_a)
        g_b = pltpu.async_copy(h2_sh.at[sidx_b], rows_b, sem_b)
        g_a.wait()
        pltpu.sync_copy(rows_a, hsrc_hbm.at[pl.ds(gbase, ga)])
        g_b.wait()
        pltpu.sync_copy(rows_b, hsrc_hbm.at[pl.ds(gbase + ga, gb)])

    return fused


def kernel(x, edge_index, e, W_in, b_in,
           W_fgn_0, b_fgn_0, root_0, b_ecc_0, W_mlp_0, b_mlp_0,
           W_fgn_1, b_fgn_1, root_1, b_ecc_1, W_mlp_1, b_mlp_1,
           W_out, b_out):
    n = x.shape[0]
    ecount = edge_index.shape[1]
    h = W_in.shape[1]

    p = jnp.arange(ecount, dtype=jnp.int32)
    perm_e = (p // _EB) * _EB + (_EB // 8) * (p % 8) + (p % _EB) // 8
    src_s = jnp.take(edge_index[0], perm_e)
    dst_s = jnp.take(edge_index[1], perm_e)
    src_s = 8 * (src_s % (n // 8)) + src_s // (n // 8)
    dst_s = 8 * (dst_s % (n // 8)) + dst_s // (n // 8)
    et = e.T

    gather = _make_gather(n, ecount, h)
    scatter = _make_scatter(n, ecount, h)
    fused = _make_scatter_update_gather(n, ecount, h)

    h1w, hr1w, hm1w = _in_proj(x, W_in, b_in,
                               root_0, b_ecc_0, W_mlp_0, b_mlp_0)
    h_src1 = gather(h1w.reshape(n, h), src_s)
    msg1w = _edge_msg(et, h_src1.reshape(ecount * h // 128, 128),
                      W_fgn_0, b_fgn_0)
    h2, h_src2 = fused(msg1w.reshape(ecount, h), dst_s, src_s,
                       h1w.reshape(n, h), hr1w.reshape(n, h),
                       hm1w.reshape(n, h))
    msg2w = _edge_msg(et, h_src2.reshape(ecount * h // 128, 128),
                      W_fgn_1, b_fgn_1)
    parts = scatter(msg2w.reshape(ecount, h), dst_s)
    return _update_out(h2.reshape(n * h // 128, 128),
                       parts.reshape(_NC * n * h // 128, 128),
                       root_1, b_ecc_1, W_mlp_1, b_mlp_1, W_out, b_out)


# speedup vs baseline: 2.0926x; 1.0396x over previous
import functools

import jax
import jax.numpy as jnp
from jax import lax
from jax.experimental import pallas as pl
from jax.experimental.pallas import tpu as pltpu
from jax.experimental.pallas import tpu_sc as plsc

_NC = 2
_NS = 16
_NW = _NC * _NS

_EB = 16000


def _store_slots(ref, val, h):
    grp = val.shape[0] // 8
    for g in range(8):
        ref[:, g * h:(g + 1) * h] = val[g * grp:(g + 1) * grp, :]


def _load_slots(val_w, h):
    return jnp.concatenate(
        [val_w[:, g * h:(g + 1) * h] for g in range(8)], axis=0)


def _in_proj_body(x_ref, w_ref, b_ref, r_ref, bc_ref, wm_ref, bm_ref,
                  o_ref, hr_ref, hm_ref):
    h = w_ref.shape[1]
    hv = (
        jnp.dot(x_ref[...], w_ref[...], preferred_element_type=jnp.float32)
        + b_ref[...]
    )
    _store_slots(o_ref, hv, h)
    hr = (
        jnp.dot(hv, r_ref[...], preferred_element_type=jnp.float32)
        + bc_ref[...]
    )
    _store_slots(hr_ref, hr, h)
    hm = jnp.maximum(
        jnp.dot(hv, wm_ref[...], preferred_element_type=jnp.float32)
        + bm_ref[...], 0.0)
    _store_slots(hm_ref, hm, h)


def _edge_msg_body(et_ref, hs_ref, wf_ref, bf_ref, o_ref):
    h = et_ref.shape[0]
    hh = h * h
    ev = et_ref[...].T
    hs = _load_slots(hs_ref[...], h)
    kern = (
        jnp.dot(ev, wf_ref[...], preferred_element_type=jnp.float32)
        + bf_ref[...]
    )
    rf = lax.broadcasted_iota(jnp.int32, (h, hh), 0)
    rj = lax.broadcasted_iota(jnp.int32, (h, hh), 1) // h
    rmat = jnp.where(rf == rj, 1.0, 0.0).astype(jnp.float32)
    hrep = jnp.dot(hs, rmat, preferred_element_type=jnp.float32)
    sj = lax.broadcasted_iota(jnp.int32, (hh, h), 0) % h
    so = lax.broadcasted_iota(jnp.int32, (hh, h), 1)
    smat = jnp.where(sj == so, 1.0, 0.0).astype(jnp.float32)
    msg = jnp.dot(kern * hrep, smat, preferred_element_type=jnp.float32)
    _store_slots(o_ref, msg, h)


def _update_out_body(h_ref, p_ref, r_ref, bc_ref, wm_ref, bm_ref,
                     wo_ref, bo_ref, o_ref):
    h = r_ref.shape[0]
    half = p_ref.shape[0] // 2
    hv = _load_slots(h_ref[...], h)
    pv = p_ref[...]
    agg = _load_slots(pv[:half] + pv[half:], h)
    hg = jnp.maximum(
        agg + jnp.dot(hv, r_ref[...], preferred_element_type=jnp.float32)
        + bc_ref[...], 0.0)
    hm = jnp.maximum(
        jnp.dot(hv, wm_ref[...], preferred_element_type=jnp.float32)
        + bm_ref[...], 0.0)
    hn = hg + hm + hv
    o_ref[...] = (
        jnp.dot(hn, wo_ref[...], preferred_element_type=jnp.float32)
        + bo_ref[...]
    )


def _full(shape):
    return pl.BlockSpec(shape, lambda i: tuple(0 for _ in shape))


def _in_proj(x, w, b, r, bc, wm, bm):
    n, d = x.shape
    h = w.shape[1]
    wn = n * h // 128
    wspec = _full((wn, 128))
    return pl.pallas_call(
        _in_proj_body,
        grid=(1,),
        in_specs=[
            _full((n, d)),
            _full((d, h)),
            _full((1, h)),
            _full((h, h)),
            _full((1, h)),
            _full((h, h)),
            _full((1, h)),
        ],
        out_specs=[wspec, wspec, wspec],
        out_shape=[jax.ShapeDtypeStruct((wn, 128), jnp.float32)] * 3,
    )(x, w, b.reshape(1, h), r, bc.reshape(1, h), wm, bm.reshape(1, h))


def _edge_msg(et, h_src_w, wf, bf):
    d_edge, ecount = et.shape
    h = d_edge
    hh = h * h
    wb = _EB * h // 128
    return pl.pallas_call(
        _edge_msg_body,
        grid=(ecount // _EB,),
        in_specs=[
            pl.BlockSpec((d_edge, _EB), lambda i: (0, i)),
            pl.BlockSpec((wb, 128), lambda i: (i, 0)),
            _full((d_edge, hh)),
            _full((1, hh)),
        ],
        out_specs=pl.BlockSpec((wb, 128), lambda i: (i, 0)),
        out_shape=jax.ShapeDtypeStruct((ecount * h // 128, 128), jnp.float32),
    )(et, h_src_w, wf, bf.reshape(1, hh))


def _update_out(h_w, parts_w, r, bc, wm, bm, wo, bo):
    h = r.shape[0]
    n = h_w.shape[0] * 128 // h
    o = wo.shape[1]
    return pl.pallas_call(
        _update_out_body,
        grid=(1,),
        in_specs=[
            _full(h_w.shape),
            _full(parts_w.shape),
            _full((h, h)),
            _full((1, h)),
            _full((h, h)),
            _full((1, h)),
            _full((h, o)),
            _full((1, o)),
        ],
        out_specs=_full((n, o)),
        out_shape=jax.ShapeDtypeStruct((n, o), jnp.float32),
    )(h_w, parts_w, r, bc.reshape(1, h), wm, bm.reshape(1, h),
      wo, bo.reshape(1, o))


@functools.cache
def _make_gather(n, ecount, h):
    epw = ecount // _NW
    mesh = plsc.VectorSubcoreMesh(core_axis_name="c", subcore_axis_name="s")

    @functools.partial(
        pl.kernel,
        out_type=jax.ShapeDtypeStruct((ecount, h), jnp.float32),
        mesh=mesh,
        compiler_params=pltpu.CompilerParams(use_tc_tiling_on_sc=False),
        scratch_types=[
            pltpu.VMEM((epw,), jnp.int32),
            pltpu.VMEM((epw, h), jnp.float32),
            pltpu.SemaphoreType.DMA,
        ],
    )
    def gather(h_hbm, idx_hbm, out_hbm, idx_v, rows_v, sem):
        wid = lax.axis_index("s") * _NC + lax.axis_index("c")
        base = wid * epw
        pltpu.sync_copy(idx_hbm.at[pl.ds(base, epw)], idx_v)
        pltpu.async_copy(h_hbm.at[idx_v], rows_v, sem).wait()
        pltpu.sync_copy(rows_v, out_hbm.at[pl.ds(base, epw)])

    return gather


@functools.cache
def _make_scatter(n, ecount, h):
    epw = ecount // _NW
    npt = n // _NS
    mesh = plsc.VectorSubcoreMesh(core_axis_name="c", subcore_axis_name="s")

    @functools.partial(
        pl.kernel,
        out_type=jax.ShapeDtypeStruct((_NC * n, h), jnp.float32),
        mesh=mesh,
        compiler_params=pltpu.CompilerParams(use_tc_tiling_on_sc=False),
        scratch_types=[
            pltpu.VMEM((epw,), jnp.int32),
            pltpu.VMEM((epw, h), jnp.float32),
            pltpu.VMEM((npt, h), jnp.float32),
            pltpu.VMEM_SHARED((n, h), jnp.float32),
            pltpu.SemaphoreType.DMA,
            pltpu.SemaphoreType.DMA,
        ],
    )
    def scatter(msg_hbm, dst_hbm, out_hbm, idx_v, rows_v, obuf, agg_sh,
                sem_i, sem_r):
        cid = lax.axis_index("c")
        sid = lax.axis_index("s")
        wid = cid * _NS + sid
        base = wid * epw

        cp_i = pltpu.async_copy(dst_hbm.at[pl.ds(base, epw)], idx_v, sem_i)
        cp_r = pltpu.async_copy(msg_hbm.at[pl.ds(base, epw)], rows_v, sem_r)

        def zero_row(i, _):
            obuf[i, :] = jnp.zeros((h,), jnp.float32)
            return 0
        lax.fori_loop(0, npt, zero_row, 0)
        pltpu.sync_copy(obuf, agg_sh.at[pl.ds(sid * npt, npt)])
        cp_i.wait()
        cp_r.wait()
        plsc.subcore_barrier()

        pltpu.sync_copy(rows_v, agg_sh.at[idx_v], add=True)
        plsc.subcore_barrier()

        pltpu.sync_copy(agg_sh.at[pl.ds(sid * npt, npt)], obuf)
        pltpu.sync_copy(obuf, out_hbm.at[pl.ds(cid * n + sid * npt, npt)])

    return scatter


@functools.cache
def _make_scatter_update_gather(n, ecount, h):
    ept = ecount // _NS
    q = ept // 5
    epw = ecount // _NW
    ga = q
    gb = epw - q
    npt = n // _NS
    mesh = plsc.VectorSubcoreMesh(core_axis_name="c", subcore_axis_name="s")

    @functools.partial(
        pl.kernel,
        out_type=(
            jax.ShapeDtypeStruct((n, h), jnp.float32),
            jax.ShapeDtypeStruct((ecount, h), jnp.float32),
        ),
        mesh=mesh,
        compiler_params=pltpu.CompilerParams(use_tc_tiling_on_sc=False),
        scratch_types=[
            pltpu.VMEM((5, q), jnp.int32),
            pltpu.VMEM((ga,), jnp.int32),
            pltpu.VMEM((gb,), jnp.int32),
            pltpu.VMEM((q, h), jnp.float32),
            pltpu.VMEM((gb, h), jnp.float32),
            pltpu.VMEM((npt, h), jnp.float32),
            pltpu.VMEM_SHARED((n, h), jnp.float32),
            pltpu.VMEM_SHARED((n, h), jnp.float32),
            pltpu.SemaphoreType.DMA,
            pltpu.SemaphoreType.DMA,
            pltpu.SemaphoreType.DMA,
            pltpu.SemaphoreType.DMA,
        ],
    )
    def fused(msg_hbm, dst_hbm, src_hbm, h_hbm, hr_hbm, hm_hbm,
              hnew_hbm, hsrc_hbm,
              idx_v, sidx_a, sidx_b, rows_a, rows_b, zbuf,
              agg_sh, h2_sh, sem_i, sem_a, sem_b, sem_g):
        cid = lax.axis_index("c")
        sid = lax.axis_index("s")
        ebase = sid * ept
        nbase = sid * npt
        gbase = (sid * _NC + cid) * epw

        cpi = [pltpu.async_copy(dst_hbm.at[pl.ds(ebase + k * q, q)],
                                idx_v.at[k], sem_i)
               for k in range(5)]
        cps_a = pltpu.async_copy(src_hbm.at[pl.ds(gbase, ga)], sidx_a, sem_g)
        cps_b = pltpu.async_copy(src_hbm.at[pl.ds(gbase + ga, gb)], sidx_b,
                                 sem_g)
        bufs = [rows_a, rows_b.at[pl.ds(0, q)]]
        sems = [sem_a, sem_b]
        cr = [pltpu.async_copy(msg_hbm.at[pl.ds(ebase, q)], bufs[0], sems[0]),
              pltpu.async_copy(msg_hbm.at[pl.ds(ebase + q, q)], bufs[1],
                               sems[1])]

        def zero_row(i, _):
            zbuf[i, :] = jnp.zeros((h,), jnp.float32)
            return 0
        lax.fori_loop(0, npt, zero_row, 0)
        pltpu.sync_copy(zbuf, agg_sh.at[pl.ds(nbase, npt)])
        for c in cpi:
            c.wait()
        plsc.subcore_barrier()

        for k in range(5):
            cr[k].wait()
            pltpu.sync_copy(bufs[k % 2], agg_sh.at[idx_v.at[k]], add=True)
            if k + 2 < 5:
                cr.append(pltpu.async_copy(
                    msg_hbm.at[pl.ds(ebase + (k + 2) * q, q)],
                    bufs[k % 2], sems[k % 2]))
        plsc.subcore_barrier()

        cu_h = pltpu.async_copy(h_hbm.at[pl.ds(nbase, npt)],
                                rows_a.at[pl.ds(0, npt)], sem_a)
        cu_r = pltpu.async_copy(hr_hbm.at[pl.ds(nbase, npt)],
                                rows_a.at[pl.ds(npt, npt)], sem_b)
        cu_m = pltpu.async_copy(hm_hbm.at[pl.ds(nbase, npt)],
                                rows_a.at[pl.ds(2 * npt, npt)], sem_i)
        pltpu.sync_copy(agg_sh.at[pl.ds(nbase, npt)],
                        rows_b.at[pl.ds(0, npt)])
        cu_h.wait()
        cu_r.wait()
        cu_m.wait()

        def upd_row(i, _):
            hg = jnp.maximum(rows_b[i, :] + rows_a[npt + i, :], 0.0)
            rows_b[npt + i, :] = hg + rows_a[2 * npt + i, :] + rows_a[i, :]
            return 0
        lax.fori_loop(0, npt, upd_row, 0)

        pltpu.sync_copy(rows_b.at[pl.ds(npt, npt)],
                        h2_sh.at[pl.ds(nbase, npt)])

        @pl.when(cid == 0)
        def _():
            pltpu.sync_copy(rows_b.at[pl.ds(npt, npt)],
                            hnew_hbm.at[pl.ds(nbase, npt)])

        cps_a.wait()
        cps_b.wait()
        plsc.subcore_barrier()

        g_a = pltpu.async_copy(h2_sh.at[sidx_a], rows_a, sem_a)
        g_b = pltpu.async_copy(h2_sh.at[sidx_b], rows_b, sem_b)
        g_a.wait()
        pltpu.sync_copy(rows_a, hsrc_hbm.at[pl.ds(gbase, ga)])
        g_b.wait()
        pltpu.sync_copy(rows_b, hsrc_hbm.at[pl.ds(gbase + ga, gb)])

    return fused


def kernel(x, edge_index, e, W_in, b_in,
           W_fgn_0, b_fgn_0, root_0, b_ecc_0, W_mlp_0, b_mlp_0,
           W_fgn_1, b_fgn_1, root_1, b_ecc_1, W_mlp_1, b_mlp_1,
           W_out, b_out):
    n = x.shape[0]
    ecount = edge_index.shape[1]
    h = W_in.shape[1]

    p = jnp.arange(ecount, dtype=jnp.int32)
    perm_e = (p // _EB) * _EB + (_EB // 8) * (p % 8) + (p % _EB) // 8
    src_s = jnp.take(edge_index[0], perm_e)
    dst_s = jnp.take(edge_index[1], perm_e)
    src_s = 8 * (src_s % (n // 8)) + src_s // (n // 8)
    dst_s = 8 * (dst_s % (n // 8)) + dst_s // (n // 8)
    et = e.T

    gather = _make_gather(n, ecount, h)
    scatter = _make_scatter(n, ecount, h)
    fused = _make_scatter_update_gather(n, ecount, h)

    h1w, hr1w, hm1w = _in_proj(x, W_in, b_in,
                               root_0, b_ecc_0, W_mlp_0, b_mlp_0)
    h_src1 = gather(h1w.reshape(n, h), src_s)
    msg1w = _edge_msg(et, h_src1.reshape(ecount * h // 128, 128),
                      W_fgn_0, b_fgn_0)
    h2, h_src2 = fused(msg1w.reshape(ecount, h), dst_s, src_s,
                       h1w.reshape(n, h), hr1w.reshape(n, h),
                       hm1w.reshape(n, h))
    msg2w = _edge_msg(et, h_src2.reshape(ecount * h // 128, 128),
                      W_fgn_1, b_fgn_1)
    parts = scatter(msg2w.reshape(ecount, h), dst_s)
    return _update_out(h2.reshape(n * h // 128, 128),
                       parts.reshape(_NC * n * h // 128, 128),
                       root_1, b_ecc_1, W_mlp_1, b_mlp_1, W_out, b_out)
